# Initial kernel scaffold; baseline (speedup 1.0000x reference)
import jax
import jax.numpy as jnp
from jax.experimental import pallas as pl


def kernel(means3D, means2D, opacities, scales, rotations, colors_precomp, viewmatrix, campos):
    raise NotImplementedError("write your pallas kernel here")


# speedup vs baseline: 1.7428x; 1.7428x over previous
import functools

import jax
import jax.numpy as jnp
from jax import lax
from jax.experimental import pallas as pl
from jax.experimental.pallas import tpu as pltpu

N = 8192
H = 32
W = 32
TOPK = 16
FX = W / (2.0 * 0.5)
FY = H / (2.0 * 0.5)
KER = 0.3
P = H * W
PB = 128
CH = 1024
RB = 1024

_INTERPRET = False


def _bfr(x):
    return x.astype(jnp.bfloat16).astype(jnp.float32)


def _prep_body(vm, mx, my, mz, sx, sy, sz, q0, q1, q2, q3,
               x2d_o, y2d_o, a_o, b_o, c_o, z_o, rad_o):
    R00, R01, R02, t0 = vm[0], vm[1], vm[2], vm[3]
    R10, R11, R12, t1 = vm[4], vm[5], vm[6], vm[7]
    R20, R21, R22, t2 = vm[8], vm[9], vm[10], vm[11]
    bR00, bR01, bR02 = _bfr(R00), _bfr(R01), _bfr(R02)
    bR10, bR11, bR12 = _bfr(R10), _bfr(R11), _bfr(R12)
    bR20, bR21, bR22 = _bfr(R20), _bfr(R21), _bfr(R22)
    bmx, bmy, bmz = _bfr(mx[...]), _bfr(my[...]), _bfr(mz[...])
    px = bmx * bR00 + bmy * bR01 + bmz * bR02 + t0
    py = bmx * bR10 + bmy * bR11 + bmz * bR12 + t1
    pz = bmx * bR20 + bmy * bR21 + bmz * bR22 + t2
    z = jnp.maximum(pz, 0.2)
    x2d_o[...] = px / z * FX + W / 2.0
    y2d_o[...] = py / z * FY + H / 2.0
    qw, qx, qy, qz = q0[...], q1[...], q2[...], q3[...]
    qn = jnp.sqrt(qw * qw + qx * qx + qy * qy + qz * qz) + 1e-8
    qw, qx, qy, qz = qw / qn, qx / qn, qy / qn, qz / qn
    r00 = 1 - 2 * (qy * qy + qz * qz)
    r01 = 2 * (qx * qy - qw * qz)
    r02 = 2 * (qx * qz + qw * qy)
    r10 = 2 * (qx * qy + qw * qz)
    r11 = 1 - 2 * (qx * qx + qz * qz)
    r12 = 2 * (qy * qz - qw * qx)
    r20 = 2 * (qx * qz - qw * qy)
    r21 = 2 * (qy * qz + qw * qx)
    r22 = 1 - 2 * (qx * qx + qy * qy)
    sxv, syv, szv = sx[...], sy[...], sz[...]
    bm00, bm01, bm02 = _bfr(r00 * sxv), _bfr(r01 * syv), _bfr(r02 * szv)
    bm10, bm11, bm12 = _bfr(r10 * sxv), _bfr(r11 * syv), _bfr(r12 * szv)
    bm20, bm21, bm22 = _bfr(r20 * sxv), _bfr(r21 * syv), _bfr(r22 * szv)
    Sxx = bm00 * bm00 + bm01 * bm01 + bm02 * bm02
    Sxy = bm00 * bm10 + bm01 * bm11 + bm02 * bm12
    Sxz = bm00 * bm20 + bm01 * bm21 + bm02 * bm22
    Syy = bm10 * bm10 + bm11 * bm11 + bm12 * bm12
    Syz = bm10 * bm20 + bm11 * bm21 + bm12 * bm22
    Szz = bm20 * bm20 + bm21 * bm21 + bm22 * bm22
    bJ00 = _bfr(FX / z)
    bJ02 = _bfr(-FX * px / (z * z))
    bJ11 = _bfr(FY / z)
    bJ12 = _bfr(-FY * py / (z * z))
    T00 = bJ00 * bR00 + bJ02 * bR20
    T01 = bJ00 * bR01 + bJ02 * bR21
    T02 = bJ00 * bR02 + bJ02 * bR22
    T10 = bJ11 * bR10 + bJ12 * bR20
    T11 = bJ11 * bR11 + bJ12 * bR21
    T12 = bJ11 * bR12 + bJ12 * bR22
    bT00, bT01, bT02 = _bfr(T00), _bfr(T01), _bfr(T02)
    bT10, bT11, bT12 = _bfr(T10), _bfr(T11), _bfr(T12)
    bSxx, bSxy, bSxz = _bfr(Sxx), _bfr(Sxy), _bfr(Sxz)
    bSyy, bSyz, bSzz = _bfr(Syy), _bfr(Syz), _bfr(Szz)
    u0 = bT00 * bSxx + bT01 * bSxy + bT02 * bSxz
    u1 = bT00 * bSxy + bT01 * bSyy + bT02 * bSyz
    u2 = bT00 * bSxz + bT01 * bSyz + bT02 * bSzz
    v0 = bT10 * bSxx + bT11 * bSxy + bT12 * bSxz
    v1 = bT10 * bSxy + bT11 * bSyy + bT12 * bSyz
    v2 = bT10 * bSxz + bT11 * bSyz + bT12 * bSzz
    bu0, bu1, bu2 = _bfr(u0), _bfr(u1), _bfr(u2)
    bv0, bv1, bv2 = _bfr(v0), _bfr(v1), _bfr(v2)
    c00 = bu0 * bT00 + bu1 * bT01 + bu2 * bT02
    c01 = bu0 * bT10 + bu1 * bT11 + bu2 * bT12
    c11 = bv0 * bT10 + bv1 * bT11 + bv2 * bT12
    a = c00 + KER
    b = c01
    c = c11 + KER
    det = a * c - b * b
    det = jnp.where(det == 0.0, 1e-8, det)
    a_o[...] = c / det
    b_o[...] = b / det
    c_o[...] = a / det
    mid = 0.5 * (a + c)
    lam = mid + jnp.sqrt(jnp.maximum(mid * mid - det, 0.1))
    rad_o[...] = jnp.ceil(3.0 * jnp.sqrt(lam))
    z_o[...] = z


def _prep(vm, cols):
    shp = jax.ShapeDtypeStruct((64, 128), jnp.float32)
    return pl.pallas_call(
        _prep_body,
        grid=(),
        in_specs=[pl.BlockSpec(memory_space=pltpu.SMEM)]
        + [pl.BlockSpec((64, 128), lambda: (0, 0))] * 10,
        out_specs=[pl.BlockSpec((64, 128), lambda: (0, 0))] * 7,
        out_shape=[shp] * 7,
        interpret=_INTERPRET,
    )(vm, *cols)


def _rank_body(zc, zr, out):
    b = pl.program_id(0)
    zi = zc[...]
    zj = zr[...]
    ii = lax.broadcasted_iota(jnp.int32, (RB, N), 0) + b * RB
    jj = lax.broadcasted_iota(jnp.int32, (RB, N), 1)
    lt = zj < zi
    eq = (zj == zi) & (jj < ii)
    cnt = jnp.sum((lt | eq).astype(jnp.float32), axis=1, keepdims=True)
    out[...] = cnt.astype(jnp.int32)


def _rank(zcol, zrow):
    return pl.pallas_call(
        _rank_body,
        grid=(N // RB,),
        in_specs=[pl.BlockSpec((RB, 1), lambda b: (b, 0)),
                  pl.BlockSpec((1, N), lambda b: (0, 0))],
        out_specs=pl.BlockSpec((RB, 1), lambda b: (b, 0)),
        out_shape=jax.ShapeDtypeStruct((N, 1), jnp.int32),
        interpret=_INTERPRET,
    )(zcol, zrow)


def _perm_body(rank, pmt, out):
    b = pl.program_id(0)
    r = rank[...]
    rr = lax.broadcasted_iota(jnp.int32, (RB, N), 0) + b * RB
    mask = (r == rr).astype(jnp.float32)
    out[...] = lax.dot_general(pmt[...], mask, (((1,), (1,)), ((), ())),
                               precision=lax.Precision.HIGHEST,
                               preferred_element_type=jnp.float32)


def _permute(rank_row, pmt):
    return pl.pallas_call(
        _perm_body,
        grid=(N // RB,),
        in_specs=[pl.BlockSpec((1, N), lambda b: (0, 0)),
                  pl.BlockSpec((16, N), lambda b: (0, 0))],
        out_specs=pl.BlockSpec((16, RB), lambda b: (0, b)),
        out_shape=jax.ShapeDtypeStruct((16, N), jnp.float32),
        interpret=_INTERPRET,
    )(rank_row, pmt)


def _raster_body(pmt, pxr, pyr, col_o, md_o, ai_o, tkw_o, tki_o, tvc_o, wgt_s):
    f32 = jnp.float32
    px = pxr[...]
    py = pyr[...]
    Tc = jnp.ones((PB, 1), f32)
    colR = jnp.zeros((PB, 1), f32)
    colG = jnp.zeros((PB, 1), f32)
    colB = jnp.zeros((PB, 1), f32)
    md = jnp.zeros((PB, 1), f32)
    ai = jnp.zeros((PB, 1), f32)
    for cidx in range(N // CH):
        sl = pl.ds(cidx * CH, CH)
        xs = pmt[0:1, sl]
        ys = pmt[1:2, sl]
        A = pmt[2:3, sl]
        Bc = pmt[3:4, sl]
        Cc = pmt[4:5, sl]
        op = pmt[5:6, sl]
        cr = pmt[6:7, sl]
        cg = pmt[7:8, sl]
        cb = pmt[8:9, sl]
        zz = pmt[9:10, sl]
        dx = px - xs
        dy = py - ys
        power = -0.5 * (A * dx * dx + Cc * dy * dy) - Bc * dx * dy
        alpha = jnp.where(power > 0.0, 0.0, op * jnp.exp(power))
        alpha = jnp.minimum(alpha, 0.99)
        t = 1.0 - alpha + 1e-10
        cp = t
        s = 1
        while s < CH:
            shifted = jnp.concatenate(
                [jnp.ones((PB, s), f32), cp[:, :CH - s]], axis=1)
            cp = cp * shifted
            s *= 2
        texcl = jnp.concatenate(
            [jnp.ones((PB, 1), f32), cp[:, :CH - 1]], axis=1) * Tc
        w = alpha * texcl
        wgt_s[:, sl] = w
        bw = _bfr(w)
        colR += jnp.sum(bw * _bfr(cr), axis=1, keepdims=True)
        colG += jnp.sum(bw * _bfr(cg), axis=1, keepdims=True)
        colB += jnp.sum(bw * _bfr(cb), axis=1, keepdims=True)
        md += jnp.sum(bw * _bfr(zz), axis=1, keepdims=True)
        ai += jnp.sum(w, axis=1, keepdims=True)
        Tc = Tc * cp[:, CH - 1:CH]
    col_o[...] = jnp.concatenate([colR, colG, colB], axis=1)
    md_o[...] = md
    ai_o[...] = ai
    order = pmt[10:11, :]
    iota = lax.broadcasted_iota(jnp.int32, (PB, N), 1)
    tkw_l = []
    tki_l = []
    for _ in range(TOPK):
        wv = wgt_s[...]
        m = jnp.max(wv, axis=1, keepdims=True)
        cand = jnp.where(wv == m, iota, 2 * N)
        sidx = jnp.min(cand, axis=1, keepdims=True)
        sel = iota == sidx
        oid = jnp.max(jnp.where(sel, order, -1.0), axis=1, keepdims=True)
        wgt_s[...] = jnp.where(sel, -1.0, wv)
        tkw_l.append(m)
        tki_l.append(oid)
    tkw = jnp.concatenate(tkw_l, axis=1)
    tki = jnp.concatenate(tki_l, axis=1)
    tkw_o[...] = tkw
    tki_o[...] = jnp.round(tki).astype(jnp.int32)
    tvc_o[...] = jnp.sum((tkw > 1.0 / 255.0).astype(jnp.int32),
                         axis=1, keepdims=True)


def _raster(pmt_s, pixx, pixy):
    f32 = jnp.float32
    outs = [jax.ShapeDtypeStruct((P, 3), f32),
            jax.ShapeDtypeStruct((P, 1), f32),
            jax.ShapeDtypeStruct((P, 1), f32),
            jax.ShapeDtypeStruct((P, TOPK), f32),
            jax.ShapeDtypeStruct((P, TOPK), jnp.int32),
            jax.ShapeDtypeStruct((P, 1), jnp.int32)]
    return pl.pallas_call(
        _raster_body,
        grid=(P // PB,),
        in_specs=[pl.BlockSpec((16, N), lambda b: (0, 0)),
                  pl.BlockSpec((PB, 1), lambda b: (b, 0)),
                  pl.BlockSpec((PB, 1), lambda b: (b, 0))],
        out_specs=[pl.BlockSpec((PB, 3), lambda b: (b, 0)),
                   pl.BlockSpec((PB, 1), lambda b: (b, 0)),
                   pl.BlockSpec((PB, 1), lambda b: (b, 0)),
                   pl.BlockSpec((PB, TOPK), lambda b: (b, 0)),
                   pl.BlockSpec((PB, TOPK), lambda b: (b, 0)),
                   pl.BlockSpec((PB, 1), lambda b: (b, 0))],
        out_shape=outs,
        scratch_shapes=[pltpu.VMEM((PB, N), f32)],
        interpret=_INTERPRET,
    )(pmt_s, pixx, pixy)


def kernel(means3D, means2D, opacities, scales, rotations, colors_precomp,
           viewmatrix, campos):
    f32 = jnp.float32
    rs = lambda v: v.reshape(64, 128)
    cols = [rs(means3D[:, 0]), rs(means3D[:, 1]), rs(means3D[:, 2]),
            rs(scales[:, 0]), rs(scales[:, 1]), rs(scales[:, 2]),
            rs(rotations[:, 0]), rs(rotations[:, 1]), rs(rotations[:, 2]),
            rs(rotations[:, 3])]
    vm = viewmatrix[:3, :4].reshape(12).astype(f32)
    x2d, y2d, cA, cB, cC, zc, rad = _prep(vm, cols)
    zflat = zc.reshape(N)
    rank = _rank(zflat.reshape(N, 1), zflat.reshape(1, N))
    pmt = jnp.stack([x2d.reshape(N), y2d.reshape(N), cA.reshape(N),
                     cB.reshape(N), cC.reshape(N), opacities[:, 0],
                     colors_precomp[:, 0], colors_precomp[:, 1],
                     colors_precomp[:, 2], zflat,
                     jnp.arange(N, dtype=f32),
                     jnp.zeros((N,), f32), jnp.zeros((N,), f32),
                     jnp.zeros((N,), f32), jnp.zeros((N,), f32),
                     jnp.zeros((N,), f32)], axis=0)
    pmt_s = _permute(rank.reshape(1, N), pmt)
    pix = jnp.arange(P, dtype=f32)
    pixx = (jnp.mod(pix, W) + 0.5).reshape(P, 1)
    pixy = (jnp.floor_divide(pix, W) + 0.5).reshape(P, 1)
    color, mdepth, aimg, tkw, tki, tvc = _raster(pmt_s, pixx, pixy)
    return (color.reshape(H, W, 3), rad.reshape(N), mdepth.reshape(H, W),
            aimg.reshape(H, W), tki, tkw, tvc.reshape(P))


# speedup vs baseline: 2.2669x; 1.3008x over previous
import functools

import jax
import jax.numpy as jnp
from jax import lax
from jax.experimental import pallas as pl
from jax.experimental.pallas import tpu as pltpu
from jax.experimental.pallas import tpu_sc as plsc

N = 8192
H = 32
W = 32
TOPK = 16
FX = W / (2.0 * 0.5)
FY = H / (2.0 * 0.5)
KER = 0.3
P = H * W
PB = 128
CH = 1024
RB = 1024

_INTERPRET = False


def _bfr(x):
    return x.astype(jnp.bfloat16).astype(jnp.float32)


def _prep_body(vm, mx, my, mz, sx, sy, sz, q0, q1, q2, q3,
               x2d_o, y2d_o, a_o, b_o, c_o, z_o, rad_o):
    R00, R01, R02, t0 = vm[0], vm[1], vm[2], vm[3]
    R10, R11, R12, t1 = vm[4], vm[5], vm[6], vm[7]
    R20, R21, R22, t2 = vm[8], vm[9], vm[10], vm[11]
    bR00, bR01, bR02 = _bfr(R00), _bfr(R01), _bfr(R02)
    bR10, bR11, bR12 = _bfr(R10), _bfr(R11), _bfr(R12)
    bR20, bR21, bR22 = _bfr(R20), _bfr(R21), _bfr(R22)
    bmx, bmy, bmz = _bfr(mx[...]), _bfr(my[...]), _bfr(mz[...])
    px = bmx * bR00 + bmy * bR01 + bmz * bR02 + t0
    py = bmx * bR10 + bmy * bR11 + bmz * bR12 + t1
    pz = bmx * bR20 + bmy * bR21 + bmz * bR22 + t2
    z = jnp.maximum(pz, 0.2)
    x2d_o[...] = px / z * FX + W / 2.0
    y2d_o[...] = py / z * FY + H / 2.0
    qw, qx, qy, qz = q0[...], q1[...], q2[...], q3[...]
    qn = jnp.sqrt(qw * qw + qx * qx + qy * qy + qz * qz) + 1e-8
    qw, qx, qy, qz = qw / qn, qx / qn, qy / qn, qz / qn
    r00 = 1 - 2 * (qy * qy + qz * qz)
    r01 = 2 * (qx * qy - qw * qz)
    r02 = 2 * (qx * qz + qw * qy)
    r10 = 2 * (qx * qy + qw * qz)
    r11 = 1 - 2 * (qx * qx + qz * qz)
    r12 = 2 * (qy * qz - qw * qx)
    r20 = 2 * (qx * qz - qw * qy)
    r21 = 2 * (qy * qz + qw * qx)
    r22 = 1 - 2 * (qx * qx + qy * qy)
    sxv, syv, szv = sx[...], sy[...], sz[...]
    bm00, bm01, bm02 = _bfr(r00 * sxv), _bfr(r01 * syv), _bfr(r02 * szv)
    bm10, bm11, bm12 = _bfr(r10 * sxv), _bfr(r11 * syv), _bfr(r12 * szv)
    bm20, bm21, bm22 = _bfr(r20 * sxv), _bfr(r21 * syv), _bfr(r22 * szv)
    Sxx = bm00 * bm00 + bm01 * bm01 + bm02 * bm02
    Sxy = bm00 * bm10 + bm01 * bm11 + bm02 * bm12
    Sxz = bm00 * bm20 + bm01 * bm21 + bm02 * bm22
    Syy = bm10 * bm10 + bm11 * bm11 + bm12 * bm12
    Syz = bm10 * bm20 + bm11 * bm21 + bm12 * bm22
    Szz = bm20 * bm20 + bm21 * bm21 + bm22 * bm22
    bJ00 = _bfr(FX / z)
    bJ02 = _bfr(-FX * px / (z * z))
    bJ11 = _bfr(FY / z)
    bJ12 = _bfr(-FY * py / (z * z))
    T00 = bJ00 * bR00 + bJ02 * bR20
    T01 = bJ00 * bR01 + bJ02 * bR21
    T02 = bJ00 * bR02 + bJ02 * bR22
    T10 = bJ11 * bR10 + bJ12 * bR20
    T11 = bJ11 * bR11 + bJ12 * bR21
    T12 = bJ11 * bR12 + bJ12 * bR22
    bT00, bT01, bT02 = _bfr(T00), _bfr(T01), _bfr(T02)
    bT10, bT11, bT12 = _bfr(T10), _bfr(T11), _bfr(T12)
    bSxx, bSxy, bSxz = _bfr(Sxx), _bfr(Sxy), _bfr(Sxz)
    bSyy, bSyz, bSzz = _bfr(Syy), _bfr(Syz), _bfr(Szz)
    u0 = bT00 * bSxx + bT01 * bSxy + bT02 * bSxz
    u1 = bT00 * bSxy + bT01 * bSyy + bT02 * bSyz
    u2 = bT00 * bSxz + bT01 * bSyz + bT02 * bSzz
    v0 = bT10 * bSxx + bT11 * bSxy + bT12 * bSxz
    v1 = bT10 * bSxy + bT11 * bSyy + bT12 * bSyz
    v2 = bT10 * bSxz + bT11 * bSyz + bT12 * bSzz
    bu0, bu1, bu2 = _bfr(u0), _bfr(u1), _bfr(u2)
    bv0, bv1, bv2 = _bfr(v0), _bfr(v1), _bfr(v2)
    c00 = bu0 * bT00 + bu1 * bT01 + bu2 * bT02
    c01 = bu0 * bT10 + bu1 * bT11 + bu2 * bT12
    c11 = bv0 * bT10 + bv1 * bT11 + bv2 * bT12
    a = c00 + KER
    b = c01
    c = c11 + KER
    det = a * c - b * b
    det = jnp.where(det == 0.0, 1e-8, det)
    a_o[...] = c / det
    b_o[...] = b / det
    c_o[...] = a / det
    mid = 0.5 * (a + c)
    lam = mid + jnp.sqrt(jnp.maximum(mid * mid - det, 0.1))
    rad_o[...] = jnp.ceil(3.0 * jnp.sqrt(lam))
    z_o[...] = z


def _prep(vm, cols):
    shp = jax.ShapeDtypeStruct((64, 128), jnp.float32)
    return pl.pallas_call(
        _prep_body,
        grid=(),
        in_specs=[pl.BlockSpec(memory_space=pltpu.SMEM)]
        + [pl.BlockSpec((64, 128), lambda: (0, 0))] * 10,
        out_specs=[pl.BlockSpec((64, 128), lambda: (0, 0))] * 7,
        out_shape=[shp] * 7,
        interpret=_INTERPRET,
    )(vm, *cols)


def _rank_body(zc, zr, out):
    b = pl.program_id(0)
    zi = zc[...]
    zj = zr[...]
    ii = lax.broadcasted_iota(jnp.int32, (RB, N), 0) + b * RB
    jj = lax.broadcasted_iota(jnp.int32, (RB, N), 1)
    lt = zj < zi
    eq = (zj == zi) & (jj < ii)
    cnt = jnp.sum((lt | eq).astype(jnp.float32), axis=1, keepdims=True)
    out[...] = cnt.astype(jnp.int32)


def _rank(zcol, zrow):
    return pl.pallas_call(
        _rank_body,
        grid=(N // RB,),
        in_specs=[pl.BlockSpec((RB, 1), lambda b: (b, 0)),
                  pl.BlockSpec((1, N), lambda b: (0, 0))],
        out_specs=pl.BlockSpec((RB, 1), lambda b: (b, 0)),
        out_shape=jax.ShapeDtypeStruct((N, 1), jnp.int32),
        interpret=_INTERPRET,
    )(zcol, zrow)


def _perm_body(rank, pmt, out):
    b = pl.program_id(0)
    r = rank[...]
    rr = lax.broadcasted_iota(jnp.int32, (RB, N), 0) + b * RB
    mask = (r == rr).astype(jnp.float32)
    out[...] = lax.dot_general(pmt[...], mask, (((1,), (1,)), ((), ())),
                               precision=lax.Precision.HIGHEST,
                               preferred_element_type=jnp.float32)


def _permute(rank_row, pmt):
    return pl.pallas_call(
        _perm_body,
        grid=(N // RB,),
        in_specs=[pl.BlockSpec((1, N), lambda b: (0, 0)),
                  pl.BlockSpec((16, N), lambda b: (0, 0))],
        out_specs=pl.BlockSpec((16, RB), lambda b: (0, b)),
        out_shape=jax.ShapeDtypeStruct((16, N), jnp.float32),
        interpret=_INTERPRET,
    )(rank_row, pmt)


def _sc_permute(rank3, pm):
    info = plsc.get_sparse_core_info()
    nc = info.num_cores
    mesh = plsc.VectorSubcoreMesh(core_axis_name="c", subcore_axis_name="s")

    @functools.partial(
        pl.kernel, mesh=mesh,
        out_type=jax.ShapeDtypeStruct((N, 128), jnp.float32),
        scratch_types=[pltpu.VMEM((2, 128), jnp.int32),
                       pltpu.VMEM((256, 128), jnp.float32),
                       pltpu.SemaphoreType.DMA],
    )
    def k(rank_hbm, pm_hbm, out_hbm, idx_v, rows_v, sem):
        wid = lax.axis_index("s") * nc + lax.axis_index("c")
        base = wid * 256
        pltpu.sync_copy(rank_hbm.at[wid], idx_v)
        pltpu.sync_copy(pm_hbm.at[pl.ds(base, 256)], rows_v)
        for j in range(2):
            pltpu.async_copy(rows_v.at[pl.ds(j * 128, 128)],
                             out_hbm.at[idx_v.at[j]], sem).wait()

    return k(rank3, pm)


def _raster_body(pmt, pxr, pyr, col_o, md_o, ai_o, tkw_o, tki_o, tvc_o, wgt_s):
    f32 = jnp.float32
    px = pxr[...]
    py = pyr[...]
    Tc = jnp.ones((PB, 1), f32)
    colR = jnp.zeros((PB, 1), f32)
    colG = jnp.zeros((PB, 1), f32)
    colB = jnp.zeros((PB, 1), f32)
    md = jnp.zeros((PB, 1), f32)
    ai = jnp.zeros((PB, 1), f32)
    for cidx in range(N // CH):
        sl = pl.ds(cidx * CH, CH)
        xs = pmt[0:1, sl]
        ys = pmt[1:2, sl]
        A = pmt[2:3, sl]
        Bc = pmt[3:4, sl]
        Cc = pmt[4:5, sl]
        op = pmt[5:6, sl]
        cr = pmt[6:7, sl]
        cg = pmt[7:8, sl]
        cb = pmt[8:9, sl]
        zz = pmt[9:10, sl]
        dx = px - xs
        dy = py - ys
        power = -0.5 * (A * dx * dx + Cc * dy * dy) - Bc * dx * dy
        alpha = jnp.where(power > 0.0, 0.0, op * jnp.exp(power))
        alpha = jnp.minimum(alpha, 0.99)
        t = 1.0 - alpha + 1e-10
        cp = t
        s = 1
        while s < CH:
            shifted = jnp.concatenate(
                [jnp.ones((PB, s), f32), cp[:, :CH - s]], axis=1)
            cp = cp * shifted
            s *= 2
        texcl = jnp.concatenate(
            [jnp.ones((PB, 1), f32), cp[:, :CH - 1]], axis=1) * Tc
        w = alpha * texcl
        wgt_s[:, sl] = w
        bw = _bfr(w)
        colR += jnp.sum(bw * _bfr(cr), axis=1, keepdims=True)
        colG += jnp.sum(bw * _bfr(cg), axis=1, keepdims=True)
        colB += jnp.sum(bw * _bfr(cb), axis=1, keepdims=True)
        md += jnp.sum(bw * _bfr(zz), axis=1, keepdims=True)
        ai += jnp.sum(w, axis=1, keepdims=True)
        Tc = Tc * cp[:, CH - 1:CH]
    col_o[...] = jnp.concatenate([colR, colG, colB], axis=1)
    md_o[...] = md
    ai_o[...] = ai
    order = pmt[10:11, :]
    iota = lax.broadcasted_iota(jnp.int32, (PB, N), 1)
    tkw_l = []
    tki_l = []
    for _ in range(TOPK):
        wv = wgt_s[...]
        m = jnp.max(wv, axis=1, keepdims=True)
        cand = jnp.where(wv == m, iota, 2 * N)
        sidx = jnp.min(cand, axis=1, keepdims=True)
        sel = iota == sidx
        oid = jnp.max(jnp.where(sel, order, -1.0), axis=1, keepdims=True)
        wgt_s[...] = jnp.where(sel, -1.0, wv)
        tkw_l.append(m)
        tki_l.append(oid)
    tkw = jnp.concatenate(tkw_l, axis=1)
    tki = jnp.concatenate(tki_l, axis=1)
    tkw_o[...] = tkw
    tki_o[...] = jnp.round(tki).astype(jnp.int32)
    tvc_o[...] = jnp.sum((tkw > 1.0 / 255.0).astype(jnp.int32),
                         axis=1, keepdims=True)


def _raster(pmt_s, pixx, pixy):
    f32 = jnp.float32
    outs = [jax.ShapeDtypeStruct((P, 3), f32),
            jax.ShapeDtypeStruct((P, 1), f32),
            jax.ShapeDtypeStruct((P, 1), f32),
            jax.ShapeDtypeStruct((P, TOPK), f32),
            jax.ShapeDtypeStruct((P, TOPK), jnp.int32),
            jax.ShapeDtypeStruct((P, 1), jnp.int32)]
    return pl.pallas_call(
        _raster_body,
        grid=(P // PB,),
        in_specs=[pl.BlockSpec((16, N), lambda b: (0, 0)),
                  pl.BlockSpec((PB, 1), lambda b: (b, 0)),
                  pl.BlockSpec((PB, 1), lambda b: (b, 0))],
        out_specs=[pl.BlockSpec((PB, 3), lambda b: (b, 0)),
                   pl.BlockSpec((PB, 1), lambda b: (b, 0)),
                   pl.BlockSpec((PB, 1), lambda b: (b, 0)),
                   pl.BlockSpec((PB, TOPK), lambda b: (b, 0)),
                   pl.BlockSpec((PB, TOPK), lambda b: (b, 0)),
                   pl.BlockSpec((PB, 1), lambda b: (b, 0))],
        out_shape=outs,
        scratch_shapes=[pltpu.VMEM((PB, N), f32)],
        interpret=_INTERPRET,
    )(pmt_s, pixx, pixy)


def kernel(means3D, means2D, opacities, scales, rotations, colors_precomp,
           viewmatrix, campos):
    f32 = jnp.float32
    rs = lambda v: v.reshape(64, 128)
    cols = [rs(means3D[:, 0]), rs(means3D[:, 1]), rs(means3D[:, 2]),
            rs(scales[:, 0]), rs(scales[:, 1]), rs(scales[:, 2]),
            rs(rotations[:, 0]), rs(rotations[:, 1]), rs(rotations[:, 2]),
            rs(rotations[:, 3])]
    vm = viewmatrix[:3, :4].reshape(12).astype(f32)
    x2d, y2d, cA, cB, cC, zc, rad = _prep(vm, cols)
    zflat = zc.reshape(N)
    rank = _rank(zflat.reshape(N, 1), zflat.reshape(1, N))
    pm = jnp.stack([x2d.reshape(N), y2d.reshape(N), cA.reshape(N),
                    cB.reshape(N), cC.reshape(N), opacities[:, 0],
                    colors_precomp[:, 0], colors_precomp[:, 1],
                    colors_precomp[:, 2], zflat,
                    jnp.arange(N, dtype=f32),
                    jnp.zeros((N,), f32)], axis=1)
    pm = jnp.concatenate([pm, jnp.zeros((N, 116), f32)], axis=1)
    pm_s = _sc_permute(rank.reshape(32, 2, 128), pm)
    pmt_s = pm_s[:, :16].T
    pix = jnp.arange(P, dtype=f32)
    pixx = (jnp.mod(pix, W) + 0.5).reshape(P, 1)
    pixy = (jnp.floor_divide(pix, W) + 0.5).reshape(P, 1)
    color, mdepth, aimg, tkw, tki, tvc = _raster(pmt_s, pixx, pixy)
    return (color.reshape(H, W, 3), rad.reshape(N), mdepth.reshape(H, W),
            aimg.reshape(H, W), tki, tkw, tvc.reshape(P))


# speedup vs baseline: 2.5528x; 1.1261x over previous
import functools

import jax
import jax.numpy as jnp
from jax import lax
from jax.experimental import pallas as pl
from jax.experimental.pallas import tpu as pltpu
from jax.experimental.pallas import tpu_sc as plsc

N = 8192
H = 32
W = 32
TOPK = 16
FX = W / (2.0 * 0.5)
FY = H / (2.0 * 0.5)
KER = 0.3
P = H * W
PB = 128
CH = 1024
RB = 1024

_INTERPRET = False


def _bfr(x):
    return x.astype(jnp.bfloat16).astype(jnp.float32)


def _prep_body(vm, mx, my, mz, sx, sy, sz, q0, q1, q2, q3,
               x2d_o, y2d_o, a_o, b_o, c_o, z_o, rad_o):
    R00, R01, R02, t0 = vm[0], vm[1], vm[2], vm[3]
    R10, R11, R12, t1 = vm[4], vm[5], vm[6], vm[7]
    R20, R21, R22, t2 = vm[8], vm[9], vm[10], vm[11]
    bR00, bR01, bR02 = _bfr(R00), _bfr(R01), _bfr(R02)
    bR10, bR11, bR12 = _bfr(R10), _bfr(R11), _bfr(R12)
    bR20, bR21, bR22 = _bfr(R20), _bfr(R21), _bfr(R22)
    bmx, bmy, bmz = _bfr(mx[...]), _bfr(my[...]), _bfr(mz[...])
    px = bmx * bR00 + bmy * bR01 + bmz * bR02 + t0
    py = bmx * bR10 + bmy * bR11 + bmz * bR12 + t1
    pz = bmx * bR20 + bmy * bR21 + bmz * bR22 + t2
    z = jnp.maximum(pz, 0.2)
    x2d_o[...] = px / z * FX + W / 2.0
    y2d_o[...] = py / z * FY + H / 2.0
    qw, qx, qy, qz = q0[...], q1[...], q2[...], q3[...]
    qn = jnp.sqrt(qw * qw + qx * qx + qy * qy + qz * qz) + 1e-8
    qw, qx, qy, qz = qw / qn, qx / qn, qy / qn, qz / qn
    r00 = 1 - 2 * (qy * qy + qz * qz)
    r01 = 2 * (qx * qy - qw * qz)
    r02 = 2 * (qx * qz + qw * qy)
    r10 = 2 * (qx * qy + qw * qz)
    r11 = 1 - 2 * (qx * qx + qz * qz)
    r12 = 2 * (qy * qz - qw * qx)
    r20 = 2 * (qx * qz - qw * qy)
    r21 = 2 * (qy * qz + qw * qx)
    r22 = 1 - 2 * (qx * qx + qy * qy)
    sxv, syv, szv = sx[...], sy[...], sz[...]
    bm00, bm01, bm02 = _bfr(r00 * sxv), _bfr(r01 * syv), _bfr(r02 * szv)
    bm10, bm11, bm12 = _bfr(r10 * sxv), _bfr(r11 * syv), _bfr(r12 * szv)
    bm20, bm21, bm22 = _bfr(r20 * sxv), _bfr(r21 * syv), _bfr(r22 * szv)
    Sxx = bm00 * bm00 + bm01 * bm01 + bm02 * bm02
    Sxy = bm00 * bm10 + bm01 * bm11 + bm02 * bm12
    Sxz = bm00 * bm20 + bm01 * bm21 + bm02 * bm22
    Syy = bm10 * bm10 + bm11 * bm11 + bm12 * bm12
    Syz = bm10 * bm20 + bm11 * bm21 + bm12 * bm22
    Szz = bm20 * bm20 + bm21 * bm21 + bm22 * bm22
    bJ00 = _bfr(FX / z)
    bJ02 = _bfr(-FX * px / (z * z))
    bJ11 = _bfr(FY / z)
    bJ12 = _bfr(-FY * py / (z * z))
    T00 = bJ00 * bR00 + bJ02 * bR20
    T01 = bJ00 * bR01 + bJ02 * bR21
    T02 = bJ00 * bR02 + bJ02 * bR22
    T10 = bJ11 * bR10 + bJ12 * bR20
    T11 = bJ11 * bR11 + bJ12 * bR21
    T12 = bJ11 * bR12 + bJ12 * bR22
    bT00, bT01, bT02 = _bfr(T00), _bfr(T01), _bfr(T02)
    bT10, bT11, bT12 = _bfr(T10), _bfr(T11), _bfr(T12)
    bSxx, bSxy, bSxz = _bfr(Sxx), _bfr(Sxy), _bfr(Sxz)
    bSyy, bSyz, bSzz = _bfr(Syy), _bfr(Syz), _bfr(Szz)
    u0 = bT00 * bSxx + bT01 * bSxy + bT02 * bSxz
    u1 = bT00 * bSxy + bT01 * bSyy + bT02 * bSyz
    u2 = bT00 * bSxz + bT01 * bSyz + bT02 * bSzz
    v0 = bT10 * bSxx + bT11 * bSxy + bT12 * bSxz
    v1 = bT10 * bSxy + bT11 * bSyy + bT12 * bSyz
    v2 = bT10 * bSxz + bT11 * bSyz + bT12 * bSzz
    bu0, bu1, bu2 = _bfr(u0), _bfr(u1), _bfr(u2)
    bv0, bv1, bv2 = _bfr(v0), _bfr(v1), _bfr(v2)
    c00 = bu0 * bT00 + bu1 * bT01 + bu2 * bT02
    c01 = bu0 * bT10 + bu1 * bT11 + bu2 * bT12
    c11 = bv0 * bT10 + bv1 * bT11 + bv2 * bT12
    a = c00 + KER
    b = c01
    c = c11 + KER
    det = a * c - b * b
    det = jnp.where(det == 0.0, 1e-8, det)
    a_o[...] = c / det
    b_o[...] = b / det
    c_o[...] = a / det
    mid = 0.5 * (a + c)
    lam = mid + jnp.sqrt(jnp.maximum(mid * mid - det, 0.1))
    rad_o[...] = jnp.ceil(3.0 * jnp.sqrt(lam))
    z_o[...] = z


def _prep(vm, cols):
    shp = jax.ShapeDtypeStruct((64, 128), jnp.float32)
    return pl.pallas_call(
        _prep_body,
        grid=(),
        in_specs=[pl.BlockSpec(memory_space=pltpu.SMEM)]
        + [pl.BlockSpec((64, 128), lambda: (0, 0))] * 10,
        out_specs=[pl.BlockSpec((64, 128), lambda: (0, 0))] * 7,
        out_shape=[shp] * 7,
        interpret=_INTERPRET,
    )(vm, *cols)


def _rank_body(zc, zr, out):
    b = pl.program_id(0)
    zi = zc[...]
    zj = zr[...]
    ii = lax.broadcasted_iota(jnp.int32, (RB, N), 0) + b * RB
    jj = lax.broadcasted_iota(jnp.int32, (RB, N), 1)
    lt = zj < zi
    eq = (zj == zi) & (jj < ii)
    cnt = jnp.sum((lt | eq).astype(jnp.float32), axis=1, keepdims=True)
    out[...] = cnt.astype(jnp.int32)


def _rank(zcol, zrow):
    return pl.pallas_call(
        _rank_body,
        grid=(N // RB,),
        in_specs=[pl.BlockSpec((RB, 1), lambda b: (b, 0)),
                  pl.BlockSpec((1, N), lambda b: (0, 0))],
        out_specs=pl.BlockSpec((RB, 1), lambda b: (b, 0)),
        out_shape=jax.ShapeDtypeStruct((N, 1), jnp.int32),
        interpret=_INTERPRET,
    )(zcol, zrow)


def _perm_body(rank, pmt, out):
    b = pl.program_id(0)
    r = rank[...]
    rr = lax.broadcasted_iota(jnp.int32, (RB, N), 0) + b * RB
    mask = (r == rr).astype(jnp.float32)
    out[...] = lax.dot_general(pmt[...], mask, (((1,), (1,)), ((), ())),
                               precision=lax.Precision.HIGHEST,
                               preferred_element_type=jnp.float32)


def _permute(rank_row, pmt):
    return pl.pallas_call(
        _perm_body,
        grid=(N // RB,),
        in_specs=[pl.BlockSpec((1, N), lambda b: (0, 0)),
                  pl.BlockSpec((16, N), lambda b: (0, 0))],
        out_specs=pl.BlockSpec((16, RB), lambda b: (0, b)),
        out_shape=jax.ShapeDtypeStruct((16, N), jnp.float32),
        interpret=_INTERPRET,
    )(rank_row, pmt)


def _sc_permute(rank3, pm):
    info = plsc.get_sparse_core_info()
    nc = info.num_cores
    mesh = plsc.VectorSubcoreMesh(core_axis_name="c", subcore_axis_name="s")

    @functools.partial(
        pl.kernel, mesh=mesh,
        out_type=jax.ShapeDtypeStruct((N, 128), jnp.float32),
        scratch_types=[pltpu.VMEM((2, 128), jnp.int32),
                       pltpu.VMEM((256, 128), jnp.float32),
                       pltpu.SemaphoreType.DMA],
    )
    def k(rank_hbm, pm_hbm, out_hbm, idx_v, rows_v, sem):
        wid = lax.axis_index("s") * nc + lax.axis_index("c")
        base = wid * 256
        pltpu.sync_copy(rank_hbm.at[wid], idx_v)
        pltpu.sync_copy(pm_hbm.at[pl.ds(base, 256)], rows_v)
        for j in range(2):
            pltpu.async_copy(rows_v.at[pl.ds(j * 128, 128)],
                             out_hbm.at[idx_v.at[j]], sem).wait()

    return k(rank3, pm)


def _raster_body(pmt, pxr, pyr, col_o, md_o, ai_o, tkw_o, tki_o, tvc_o, wgt_s):
    f32 = jnp.float32
    px = pxr[...]
    py = pyr[...]
    Tc = jnp.ones((PB, 1), f32)
    colR = jnp.zeros((PB, 1), f32)
    colG = jnp.zeros((PB, 1), f32)
    colB = jnp.zeros((PB, 1), f32)
    md = jnp.zeros((PB, 1), f32)
    ai = jnp.zeros((PB, 1), f32)
    for cidx in range(N // CH):
        sl = pl.ds(cidx * CH, CH)
        xs = pmt[0:1, sl]
        ys = pmt[1:2, sl]
        A = pmt[2:3, sl]
        Bc = pmt[3:4, sl]
        Cc = pmt[4:5, sl]
        op = pmt[5:6, sl]
        cr = pmt[6:7, sl]
        cg = pmt[7:8, sl]
        cb = pmt[8:9, sl]
        zz = pmt[9:10, sl]
        dx = px - xs
        dy = py - ys
        power = -0.5 * (A * dx * dx + Cc * dy * dy) - Bc * dx * dy
        alpha = jnp.where(power > 0.0, 0.0, op * jnp.exp(power))
        alpha = jnp.minimum(alpha, 0.99)
        t = 1.0 - alpha + 1e-10
        cp = t
        s = 1
        while s < CH:
            shifted = jnp.concatenate(
                [jnp.ones((PB, s), f32), cp[:, :CH - s]], axis=1)
            cp = cp * shifted
            s *= 2
        texcl = jnp.concatenate(
            [jnp.ones((PB, 1), f32), cp[:, :CH - 1]], axis=1) * Tc
        w = alpha * texcl
        wgt_s[:, sl] = w
        bw = _bfr(w)
        colR += jnp.sum(bw * _bfr(cr), axis=1, keepdims=True)
        colG += jnp.sum(bw * _bfr(cg), axis=1, keepdims=True)
        colB += jnp.sum(bw * _bfr(cb), axis=1, keepdims=True)
        md += jnp.sum(bw * _bfr(zz), axis=1, keepdims=True)
        ai += jnp.sum(w, axis=1, keepdims=True)
        Tc = Tc * cp[:, CH - 1:CH]
    col_o[...] = jnp.concatenate([colR, colG, colB], axis=1)
    md_o[...] = md
    ai_o[...] = ai
    order = pmt[10:11, :]
    iota = lax.broadcasted_iota(jnp.int32, (PB, N), 1)
    pk = iota * N + jnp.round(order).astype(jnp.int32)
    tkw_l = []
    tki_l = []
    for _ in range(TOPK):
        wv = wgt_s[...]
        m = jnp.max(wv, axis=1, keepdims=True)
        cand = jnp.where(wv == m, pk, jnp.int32(2147483647))
        pmin = jnp.min(cand, axis=1, keepdims=True)
        wgt_s[...] = jnp.where(pk == pmin, -1.0, wv)
        tkw_l.append(m)
        tki_l.append(pmin)
    tkw = jnp.concatenate(tkw_l, axis=1)
    tki = jnp.concatenate(tki_l, axis=1)
    tkw_o[...] = tkw
    tki_o[...] = jnp.remainder(tki, N)
    tvc_o[...] = jnp.sum((tkw > 1.0 / 255.0).astype(jnp.int32),
                         axis=1, keepdims=True)


def _raster(pmt_s, pixx, pixy):
    f32 = jnp.float32
    outs = [jax.ShapeDtypeStruct((P, 3), f32),
            jax.ShapeDtypeStruct((P, 1), f32),
            jax.ShapeDtypeStruct((P, 1), f32),
            jax.ShapeDtypeStruct((P, TOPK), f32),
            jax.ShapeDtypeStruct((P, TOPK), jnp.int32),
            jax.ShapeDtypeStruct((P, 1), jnp.int32)]
    return pl.pallas_call(
        _raster_body,
        grid=(P // PB,),
        in_specs=[pl.BlockSpec((16, N), lambda b: (0, 0)),
                  pl.BlockSpec((PB, 1), lambda b: (b, 0)),
                  pl.BlockSpec((PB, 1), lambda b: (b, 0))],
        out_specs=[pl.BlockSpec((PB, 3), lambda b: (b, 0)),
                   pl.BlockSpec((PB, 1), lambda b: (b, 0)),
                   pl.BlockSpec((PB, 1), lambda b: (b, 0)),
                   pl.BlockSpec((PB, TOPK), lambda b: (b, 0)),
                   pl.BlockSpec((PB, TOPK), lambda b: (b, 0)),
                   pl.BlockSpec((PB, 1), lambda b: (b, 0))],
        out_shape=outs,
        scratch_shapes=[pltpu.VMEM((PB, N), f32)],
        interpret=_INTERPRET,
    )(pmt_s, pixx, pixy)


def kernel(means3D, means2D, opacities, scales, rotations, colors_precomp,
           viewmatrix, campos):
    f32 = jnp.float32
    rs = lambda v: v.reshape(64, 128)
    cols = [rs(means3D[:, 0]), rs(means3D[:, 1]), rs(means3D[:, 2]),
            rs(scales[:, 0]), rs(scales[:, 1]), rs(scales[:, 2]),
            rs(rotations[:, 0]), rs(rotations[:, 1]), rs(rotations[:, 2]),
            rs(rotations[:, 3])]
    vm = viewmatrix[:3, :4].reshape(12).astype(f32)
    x2d, y2d, cA, cB, cC, zc, rad = _prep(vm, cols)
    zflat = zc.reshape(N)
    rank = _rank(zflat.reshape(N, 1), zflat.reshape(1, N))
    pm = jnp.stack([x2d.reshape(N), y2d.reshape(N), cA.reshape(N),
                    cB.reshape(N), cC.reshape(N), opacities[:, 0],
                    colors_precomp[:, 0], colors_precomp[:, 1],
                    colors_precomp[:, 2], zflat,
                    jnp.arange(N, dtype=f32),
                    jnp.zeros((N,), f32)], axis=1)
    pm = jnp.concatenate([pm, jnp.zeros((N, 116), f32)], axis=1)
    pm_s = _sc_permute(rank.reshape(32, 2, 128), pm)
    pmt_s = pm_s[:, :16].T
    pix = jnp.arange(P, dtype=f32)
    pixx = (jnp.mod(pix, W) + 0.5).reshape(P, 1)
    pixy = (jnp.floor_divide(pix, W) + 0.5).reshape(P, 1)
    color, mdepth, aimg, tkw, tki, tvc = _raster(pmt_s, pixx, pixy)
    return (color.reshape(H, W, 3), rad.reshape(N), mdepth.reshape(H, W),
            aimg.reshape(H, W), tki, tkw, tvc.reshape(P))


# speedup vs baseline: 2.5536x; 1.0003x over previous
import functools

import jax
import jax.numpy as jnp
from jax import lax
from jax.experimental import pallas as pl
from jax.experimental.pallas import tpu as pltpu
from jax.experimental.pallas import tpu_sc as plsc

N = 8192
H = 32
W = 32
TOPK = 16
FX = W / (2.0 * 0.5)
FY = H / (2.0 * 0.5)
KER = 0.3
P = H * W
PB = 128
CH = 1024
RB = 1024

_INTERPRET = False


def _bfr(x):
    return x.astype(jnp.bfloat16).astype(jnp.float32)


def _prep_body(vm, mx, my, mz, sx, sy, sz, q0, q1, q2, q3,
               x2d_o, y2d_o, a_o, b_o, c_o, z_o, rad_o):
    R00, R01, R02, t0 = vm[0], vm[1], vm[2], vm[3]
    R10, R11, R12, t1 = vm[4], vm[5], vm[6], vm[7]
    R20, R21, R22, t2 = vm[8], vm[9], vm[10], vm[11]
    bR00, bR01, bR02 = _bfr(R00), _bfr(R01), _bfr(R02)
    bR10, bR11, bR12 = _bfr(R10), _bfr(R11), _bfr(R12)
    bR20, bR21, bR22 = _bfr(R20), _bfr(R21), _bfr(R22)
    bmx, bmy, bmz = _bfr(mx[...]), _bfr(my[...]), _bfr(mz[...])
    px = bmx * bR00 + bmy * bR01 + bmz * bR02 + t0
    py = bmx * bR10 + bmy * bR11 + bmz * bR12 + t1
    pz = bmx * bR20 + bmy * bR21 + bmz * bR22 + t2
    z = jnp.maximum(pz, 0.2)
    x2d_o[...] = px / z * FX + W / 2.0
    y2d_o[...] = py / z * FY + H / 2.0
    qw, qx, qy, qz = q0[...], q1[...], q2[...], q3[...]
    qn = jnp.sqrt(qw * qw + qx * qx + qy * qy + qz * qz) + 1e-8
    qw, qx, qy, qz = qw / qn, qx / qn, qy / qn, qz / qn
    r00 = 1 - 2 * (qy * qy + qz * qz)
    r01 = 2 * (qx * qy - qw * qz)
    r02 = 2 * (qx * qz + qw * qy)
    r10 = 2 * (qx * qy + qw * qz)
    r11 = 1 - 2 * (qx * qx + qz * qz)
    r12 = 2 * (qy * qz - qw * qx)
    r20 = 2 * (qx * qz - qw * qy)
    r21 = 2 * (qy * qz + qw * qx)
    r22 = 1 - 2 * (qx * qx + qy * qy)
    sxv, syv, szv = sx[...], sy[...], sz[...]
    bm00, bm01, bm02 = _bfr(r00 * sxv), _bfr(r01 * syv), _bfr(r02 * szv)
    bm10, bm11, bm12 = _bfr(r10 * sxv), _bfr(r11 * syv), _bfr(r12 * szv)
    bm20, bm21, bm22 = _bfr(r20 * sxv), _bfr(r21 * syv), _bfr(r22 * szv)
    Sxx = bm00 * bm00 + bm01 * bm01 + bm02 * bm02
    Sxy = bm00 * bm10 + bm01 * bm11 + bm02 * bm12
    Sxz = bm00 * bm20 + bm01 * bm21 + bm02 * bm22
    Syy = bm10 * bm10 + bm11 * bm11 + bm12 * bm12
    Syz = bm10 * bm20 + bm11 * bm21 + bm12 * bm22
    Szz = bm20 * bm20 + bm21 * bm21 + bm22 * bm22
    bJ00 = _bfr(FX / z)
    bJ02 = _bfr(-FX * px / (z * z))
    bJ11 = _bfr(FY / z)
    bJ12 = _bfr(-FY * py / (z * z))
    T00 = bJ00 * bR00 + bJ02 * bR20
    T01 = bJ00 * bR01 + bJ02 * bR21
    T02 = bJ00 * bR02 + bJ02 * bR22
    T10 = bJ11 * bR10 + bJ12 * bR20
    T11 = bJ11 * bR11 + bJ12 * bR21
    T12 = bJ11 * bR12 + bJ12 * bR22
    bT00, bT01, bT02 = _bfr(T00), _bfr(T01), _bfr(T02)
    bT10, bT11, bT12 = _bfr(T10), _bfr(T11), _bfr(T12)
    bSxx, bSxy, bSxz = _bfr(Sxx), _bfr(Sxy), _bfr(Sxz)
    bSyy, bSyz, bSzz = _bfr(Syy), _bfr(Syz), _bfr(Szz)
    u0 = bT00 * bSxx + bT01 * bSxy + bT02 * bSxz
    u1 = bT00 * bSxy + bT01 * bSyy + bT02 * bSyz
    u2 = bT00 * bSxz + bT01 * bSyz + bT02 * bSzz
    v0 = bT10 * bSxx + bT11 * bSxy + bT12 * bSxz
    v1 = bT10 * bSxy + bT11 * bSyy + bT12 * bSyz
    v2 = bT10 * bSxz + bT11 * bSyz + bT12 * bSzz
    bu0, bu1, bu2 = _bfr(u0), _bfr(u1), _bfr(u2)
    bv0, bv1, bv2 = _bfr(v0), _bfr(v1), _bfr(v2)
    c00 = bu0 * bT00 + bu1 * bT01 + bu2 * bT02
    c01 = bu0 * bT10 + bu1 * bT11 + bu2 * bT12
    c11 = bv0 * bT10 + bv1 * bT11 + bv2 * bT12
    a = c00 + KER
    b = c01
    c = c11 + KER
    det = a * c - b * b
    det = jnp.where(det == 0.0, 1e-8, det)
    a_o[...] = c / det
    b_o[...] = b / det
    c_o[...] = a / det
    mid = 0.5 * (a + c)
    lam = mid + jnp.sqrt(jnp.maximum(mid * mid - det, 0.1))
    rad_o[...] = jnp.ceil(3.0 * jnp.sqrt(lam))
    z_o[...] = z


def _prep(vm, cols):
    shp = jax.ShapeDtypeStruct((64, 128), jnp.float32)
    return pl.pallas_call(
        _prep_body,
        grid=(),
        in_specs=[pl.BlockSpec(memory_space=pltpu.SMEM)]
        + [pl.BlockSpec((64, 128), lambda: (0, 0))] * 10,
        out_specs=[pl.BlockSpec((64, 128), lambda: (0, 0))] * 7,
        out_shape=[shp] * 7,
        interpret=_INTERPRET,
    )(vm, *cols)


def _rank_body(zc, zr, out):
    b = pl.program_id(0)
    zi = zc[...]
    zj = zr[...]
    ii = lax.broadcasted_iota(jnp.int32, (RB, N), 0) + b * RB
    jj = lax.broadcasted_iota(jnp.int32, (RB, N), 1)
    lt = zj < zi
    eq = (zj == zi) & (jj < ii)
    cnt = jnp.sum((lt | eq).astype(jnp.float32), axis=1, keepdims=True)
    out[...] = cnt.astype(jnp.int32)


def _rank(zcol, zrow):
    return pl.pallas_call(
        _rank_body,
        grid=(N // RB,),
        in_specs=[pl.BlockSpec((RB, 1), lambda b: (b, 0)),
                  pl.BlockSpec((1, N), lambda b: (0, 0))],
        out_specs=pl.BlockSpec((RB, 1), lambda b: (b, 0)),
        out_shape=jax.ShapeDtypeStruct((N, 1), jnp.int32),
        interpret=_INTERPRET,
    )(zcol, zrow)


def _perm_body(rank, pmt, out):
    b = pl.program_id(0)
    r = rank[...]
    rr = lax.broadcasted_iota(jnp.int32, (RB, N), 0) + b * RB
    mask = (r == rr).astype(jnp.float32)
    out[...] = lax.dot_general(pmt[...], mask, (((1,), (1,)), ((), ())),
                               precision=lax.Precision.HIGHEST,
                               preferred_element_type=jnp.float32)


def _permute(rank_row, pmt):
    return pl.pallas_call(
        _perm_body,
        grid=(N // RB,),
        in_specs=[pl.BlockSpec((1, N), lambda b: (0, 0)),
                  pl.BlockSpec((16, N), lambda b: (0, 0))],
        out_specs=pl.BlockSpec((16, RB), lambda b: (0, b)),
        out_shape=jax.ShapeDtypeStruct((16, N), jnp.float32),
        interpret=_INTERPRET,
    )(rank_row, pmt)


def _sc_permute(rank3, pm):
    info = plsc.get_sparse_core_info()
    nc = info.num_cores
    mesh = plsc.VectorSubcoreMesh(core_axis_name="c", subcore_axis_name="s")

    @functools.partial(
        pl.kernel, mesh=mesh,
        out_type=jax.ShapeDtypeStruct((N, 128), jnp.float32),
        scratch_types=[pltpu.VMEM((2, 128), jnp.int32),
                       pltpu.VMEM((256, 128), jnp.float32),
                       pltpu.SemaphoreType.DMA],
    )
    def k(rank_hbm, pm_hbm, out_hbm, idx_v, rows_v, sem):
        wid = lax.axis_index("s") * nc + lax.axis_index("c")
        base = wid * 256
        pltpu.sync_copy(rank_hbm.at[wid], idx_v)
        pltpu.sync_copy(pm_hbm.at[pl.ds(base, 256)], rows_v)
        for j in range(2):
            pltpu.async_copy(rows_v.at[pl.ds(j * 128, 128)],
                             out_hbm.at[idx_v.at[j]], sem).wait()

    return k(rank3, pm)


def _raster_body(pmt, pxr, pyr, col_o, md_o, ai_o, tkw_o, tki_o, tvc_o, wgt_s):
    f32 = jnp.float32
    px = pxr[...]
    py = pyr[...]
    Tc = jnp.ones((PB, 1), f32)
    colR = jnp.zeros((PB, 1), f32)
    colG = jnp.zeros((PB, 1), f32)
    colB = jnp.zeros((PB, 1), f32)
    md = jnp.zeros((PB, 1), f32)
    ai = jnp.zeros((PB, 1), f32)
    for cidx in range(N // CH):
        sl = pl.ds(cidx * CH, CH)
        xs = pmt[0:1, sl]
        ys = pmt[1:2, sl]
        A = pmt[2:3, sl]
        Bc = pmt[3:4, sl]
        Cc = pmt[4:5, sl]
        op = pmt[5:6, sl]
        cr = pmt[6:7, sl]
        cg = pmt[7:8, sl]
        cb = pmt[8:9, sl]
        zz = pmt[9:10, sl]
        dx = px - xs
        dy = py - ys
        power = -0.5 * (A * dx * dx + Cc * dy * dy) - Bc * dx * dy
        alpha = jnp.where(power > 0.0, 0.0, op * jnp.exp(power))
        alpha = jnp.minimum(alpha, 0.99)
        t = 1.0 - alpha + 1e-10
        cp = t
        s = 1
        while s < CH:
            shifted = jnp.concatenate(
                [jnp.ones((PB, s), f32), cp[:, :CH - s]], axis=1)
            cp = cp * shifted
            s *= 2
        texcl = jnp.concatenate(
            [jnp.ones((PB, 1), f32), cp[:, :CH - 1]], axis=1) * Tc
        w = alpha * texcl
        wgt_s[:, sl] = w
        bw = _bfr(w)
        colR += jnp.sum(bw * _bfr(cr), axis=1, keepdims=True)
        colG += jnp.sum(bw * _bfr(cg), axis=1, keepdims=True)
        colB += jnp.sum(bw * _bfr(cb), axis=1, keepdims=True)
        md += jnp.sum(bw * _bfr(zz), axis=1, keepdims=True)
        ai += jnp.sum(w, axis=1, keepdims=True)
        Tc = Tc * cp[:, CH - 1:CH]
    col_o[...] = jnp.concatenate([colR, colG, colB], axis=1)
    md_o[...] = md
    ai_o[...] = ai
    order = pmt[10:11, :]
    iota = lax.broadcasted_iota(jnp.int32, (PB, N), 1)
    pk = iota * N + jnp.round(order).astype(jnp.int32)
    tkw_l = []
    tki_l = []
    wv = wgt_s[...]
    m = jnp.max(wv, axis=1, keepdims=True)
    for k in range(TOPK):
        cand = jnp.where(wv == m, pk, jnp.int32(2147483647))
        pmin = jnp.min(cand, axis=1, keepdims=True)
        tkw_l.append(m)
        tki_l.append(pmin)
        if k < TOPK - 1:
            wv = jnp.where(pk == pmin, -1.0, wv)
            m = jnp.max(wv, axis=1, keepdims=True)
    tkw = jnp.concatenate(tkw_l, axis=1)
    tki = jnp.concatenate(tki_l, axis=1)
    tkw_o[...] = tkw
    tki_o[...] = jnp.remainder(tki, N)
    tvc_o[...] = jnp.sum((tkw > 1.0 / 255.0).astype(jnp.int32),
                         axis=1, keepdims=True)


def _raster(pmt_s, pixx, pixy):
    f32 = jnp.float32
    outs = [jax.ShapeDtypeStruct((P, 3), f32),
            jax.ShapeDtypeStruct((P, 1), f32),
            jax.ShapeDtypeStruct((P, 1), f32),
            jax.ShapeDtypeStruct((P, TOPK), f32),
            jax.ShapeDtypeStruct((P, TOPK), jnp.int32),
            jax.ShapeDtypeStruct((P, 1), jnp.int32)]
    return pl.pallas_call(
        _raster_body,
        grid=(P // PB,),
        in_specs=[pl.BlockSpec((16, N), lambda b: (0, 0)),
                  pl.BlockSpec((PB, 1), lambda b: (b, 0)),
                  pl.BlockSpec((PB, 1), lambda b: (b, 0))],
        out_specs=[pl.BlockSpec((PB, 3), lambda b: (b, 0)),
                   pl.BlockSpec((PB, 1), lambda b: (b, 0)),
                   pl.BlockSpec((PB, 1), lambda b: (b, 0)),
                   pl.BlockSpec((PB, TOPK), lambda b: (b, 0)),
                   pl.BlockSpec((PB, TOPK), lambda b: (b, 0)),
                   pl.BlockSpec((PB, 1), lambda b: (b, 0))],
        out_shape=outs,
        scratch_shapes=[pltpu.VMEM((PB, N), f32)],
        interpret=_INTERPRET,
    )(pmt_s, pixx, pixy)


def kernel(means3D, means2D, opacities, scales, rotations, colors_precomp,
           viewmatrix, campos):
    f32 = jnp.float32
    rs = lambda v: v.reshape(64, 128)
    cols = [rs(means3D[:, 0]), rs(means3D[:, 1]), rs(means3D[:, 2]),
            rs(scales[:, 0]), rs(scales[:, 1]), rs(scales[:, 2]),
            rs(rotations[:, 0]), rs(rotations[:, 1]), rs(rotations[:, 2]),
            rs(rotations[:, 3])]
    vm = viewmatrix[:3, :4].reshape(12).astype(f32)
    x2d, y2d, cA, cB, cC, zc, rad = _prep(vm, cols)
    zflat = zc.reshape(N)
    rank = _rank(zflat.reshape(N, 1), zflat.reshape(1, N))
    pm = jnp.stack([x2d.reshape(N), y2d.reshape(N), cA.reshape(N),
                    cB.reshape(N), cC.reshape(N), opacities[:, 0],
                    colors_precomp[:, 0], colors_precomp[:, 1],
                    colors_precomp[:, 2], zflat,
                    jnp.arange(N, dtype=f32),
                    jnp.zeros((N,), f32)], axis=1)
    pm = jnp.concatenate([pm, jnp.zeros((N, 116), f32)], axis=1)
    pm_s = _sc_permute(rank.reshape(32, 2, 128), pm)
    pmt_s = pm_s[:, :16].T
    pix = jnp.arange(P, dtype=f32)
    pixx = (jnp.mod(pix, W) + 0.5).reshape(P, 1)
    pixy = (jnp.floor_divide(pix, W) + 0.5).reshape(P, 1)
    color, mdepth, aimg, tkw, tki, tvc = _raster(pmt_s, pixx, pixy)
    return (color.reshape(H, W, 3), rad.reshape(N), mdepth.reshape(H, W),
            aimg.reshape(H, W), tki, tkw, tvc.reshape(P))
